# baseline (speedup 1.0000x reference)
import jax
import jax.numpy as jnp
from jax import lax
from jax.experimental import pallas as pl
from jax.experimental.pallas import tpu as pltpu
from jax.experimental.pallas import tpu_sc as plsc

B = 16384
F = 26
D = 16
NC, NS, L = 2, 16, 16
NW = NC * NS
RW = B // NW
CH = 64
NCH = RW // CH
G = CH // L
IPC = F * CH
KB = IPC // 128
NCOMP = IPC // L


def _fm_body(idx_hbm, val_hbm, emb_hbm, w_hbm, b_hbm, out_hbm,
             idx_s, val_s, idx_l, rows_v, wv_v, out_v, bv_v, sem_e, sem_w):
    wid = lax.axis_index("s") * NC + lax.axis_index("c")
    pltpu.sync_copy(b_hbm, bv_v)
    iota = lax.iota(jnp.int32, L)
    i26 = iota * F

    def chunk_body(c, carry):
        base = wid * RW + c * CH
        pltpu.sync_copy(idx_hbm.at[pl.ds(base, CH)], idx_s)
        pltpu.sync_copy(val_hbm.at[pl.ds(base, CH)], val_s)
        for m in range(NCOMP):
            p = m * L
            pv = iota + p
            ridx = pv // F
            cidx = pv - ridx * F
            v = plsc.load_gather(idx_s, [ridx, cidx])
            idx_l[p // 128, pl.ds(p % 128, L)] = v
        cps = []
        for k in range(KB):
            cps.append(pltpu.async_copy(
                emb_hbm.at[idx_l.at[k]], rows_v.at[pl.ds(k * 128, 128)],
                sem_e))
        for k in range(KB):
            cps.append(pltpu.async_copy(
                w_hbm.at[idx_l.at[k]], wv_v.at[pl.ds(k * 128, 128)], sem_w))
        for cp in cps:
            cp.wait()
        bvec = bv_v[...]
        for g in range(G):
            rows16 = iota + g * L
            lanes = [i26 + (g * L * F + j) for j in range(F)]
            tv = [plsc.load_gather(val_s, [rows16, iota * 0 + j])
                  for j in range(F)]
            lin = bvec
            for j in range(F):
                lin = lin + tv[j] * plsc.load_gather(wv_v, [lanes[j]])

            def d_body(d, acc, lanes=lanes, tv=tv):
                dvec = jnp.full((L,), d, dtype=jnp.int32)
                s = jnp.zeros((L,), jnp.float32)
                for j in range(F):
                    e = plsc.load_gather(rows_v, [lanes[j], dvec])
                    t = tv[j] * e
                    s = s + t
                    acc = acc - t * t
                return acc + s * s

            acc = lax.fori_loop(0, D, d_body, jnp.zeros((L,), jnp.float32))
            out_v[pl.ds(c * CH + g * L, L)] = lin + 0.5 * acc
        return carry

    lax.fori_loop(0, NCH, chunk_body, 0)
    pltpu.sync_copy(out_v, out_hbm.at[pl.ds(wid * RW, RW)])


_MESH = plsc.VectorSubcoreMesh(
    core_axis_name="c", subcore_axis_name="s",
    num_cores=NC, num_subcores=NS)

_FM = pl.kernel(
    _fm_body,
    out_type=jax.ShapeDtypeStruct((B,), jnp.float32),
    mesh=_MESH,
    compiler_params=pltpu.CompilerParams(
        needs_layout_passes=False, use_tc_tiling_on_sc=False),
    scratch_types=[
        pltpu.VMEM((CH, 128), jnp.int32),
        pltpu.VMEM((CH, 128), jnp.float32),
        pltpu.VMEM((KB, 128), jnp.int32),
        pltpu.VMEM((IPC, D), jnp.float32),
        pltpu.VMEM((IPC,), jnp.float32),
        pltpu.VMEM((RW,), jnp.float32),
        pltpu.VMEM((L,), jnp.float32),
        pltpu.SemaphoreType.DMA,
        pltpu.SemaphoreType.DMA,
    ],
)


def kernel(feat_idx, feat_val, feature_embed, linear_w, linear_b):
    idxp = jnp.pad(feat_idx.astype(jnp.int32), ((0, 0), (0, 128 - F)))
    valp = jnp.pad(feat_val, ((0, 0), (0, 128 - F)))
    bvec = jnp.broadcast_to(linear_b.astype(jnp.float32), (L,))
    return _FM(idxp, valp, feature_embed, linear_w, bvec)


# speedup vs baseline: 1.0209x; 1.0209x over previous
import jax
import jax.numpy as jnp
from jax import lax
from jax.experimental import pallas as pl
from jax.experimental.pallas import tpu as pltpu
from jax.experimental.pallas import tpu_sc as plsc

B = 16384
F = 26
D = 16
NC, NS, L = 2, 16, 16
NW = NC * NS
RW = B // NW
CH = 64
NCH = RW // CH
NT = NCH // 2
G = CH // L
IPC = F * CH
KB = IPC // 128
NCHUNKS = B // CH


def _fm_body(idx_hbm, val_hbm, emb_hbm, w_hbm, b_hbm, out_hbm,
             idx_l0, idx_l1, val_s0, val_s1, rows_0, rows_1,
             wv_0, wv_1, out_v, bv_v, sem0, sem1):
    wid = lax.axis_index("s") * NC + lax.axis_index("c")
    cid0 = wid * NCH
    pltpu.sync_copy(b_hbm, bv_v)
    iota = lax.iota(jnp.int32, L)
    i26 = iota * F
    bufs = ((idx_l0, val_s0, rows_0, wv_0, sem0),
            (idx_l1, val_s1, rows_1, wv_1, sem1))

    def stage(cid, buf):
        idx_l, val_s, rows_v, wv_v, sem = buf
        pltpu.sync_copy(idx_hbm.at[cid], idx_l)
        pltpu.sync_copy(val_hbm.at[cid], val_s)
        for k in range(KB):
            pltpu.async_copy(
                emb_hbm.at[idx_l.at[k]], rows_v.at[pl.ds(k * 128, 128)], sem)
            pltpu.async_copy(
                w_hbm.at[idx_l.at[k]], wv_v.at[pl.ds(k * 128, 128)], sem)

    def wait_streams(buf):
        idx_l, val_s, rows_v, wv_v, sem = buf
        for k in range(KB):
            pltpu.make_async_copy(
                emb_hbm.at[idx_l.at[k]], rows_v.at[pl.ds(k * 128, 128)],
                sem).wait()
            pltpu.make_async_copy(
                w_hbm.at[idx_l.at[k]], wv_v.at[pl.ds(k * 128, 128)],
                sem).wait()

    def compute(c, buf):
        _, val_s, rows_v, wv_v, _ = buf
        bvec = bv_v[...]
        for g in range(G):
            base = g * L * F
            tv = [plsc.load_gather(val_s, [i26 + (base + j)])
                  for j in range(F)]
            l0 = bvec
            l1 = bvec * 0.0
            for j in range(F):
                wj = plsc.load_gather(wv_v, [i26 + (base + j)])
                if j % 2 == 0:
                    l0 = l0 + tv[j] * wj
                else:
                    l1 = l1 + tv[j] * wj
            lin = l0 + l1

            def d_body(d, acc, base=base, tv=tv):
                dvec = jnp.full((L,), d, dtype=jnp.int32)
                z = jnp.zeros((L,), jnp.float32)
                s0, s1, q0, q1 = z, z, z, z
                for j in range(F):
                    e = plsc.load_gather(rows_v, [i26 + (base + j), dvec])
                    t = tv[j] * e
                    if j % 2 == 0:
                        s0 = s0 + t
                        q0 = q0 + t * t
                    else:
                        s1 = s1 + t
                        q1 = q1 + t * t
                s = s0 + s1
                return acc + (s * s - (q0 + q1))

            acc = lax.fori_loop(0, D, d_body, jnp.zeros((L,), jnp.float32))
            out_v[pl.ds(c * CH + g * L, L)] = lin + 0.5 * acc
        return

    stage(cid0, bufs[0])

    def body(t, carry):
        c0 = 2 * t
        stage(cid0 + c0 + 1, bufs[1])
        wait_streams(bufs[0])
        compute(c0, bufs[0])

        @pl.when(t < NT - 1)
        def _():
            stage(cid0 + c0 + 2, bufs[0])

        wait_streams(bufs[1])
        compute(c0 + 1, bufs[1])
        return carry

    lax.fori_loop(0, NT, body, 0)
    pltpu.sync_copy(out_v, out_hbm.at[pl.ds(wid * RW, RW)])


_MESH = plsc.VectorSubcoreMesh(
    core_axis_name="c", subcore_axis_name="s",
    num_cores=NC, num_subcores=NS)

_FM = pl.kernel(
    _fm_body,
    out_type=jax.ShapeDtypeStruct((B,), jnp.float32),
    mesh=_MESH,
    compiler_params=pltpu.CompilerParams(
        needs_layout_passes=False, use_tc_tiling_on_sc=False),
    scratch_types=[
        pltpu.VMEM((KB, 128), jnp.int32),
        pltpu.VMEM((KB, 128), jnp.int32),
        pltpu.VMEM((IPC,), jnp.float32),
        pltpu.VMEM((IPC,), jnp.float32),
        pltpu.VMEM((IPC, D), jnp.float32),
        pltpu.VMEM((IPC, D), jnp.float32),
        pltpu.VMEM((IPC,), jnp.float32),
        pltpu.VMEM((IPC,), jnp.float32),
        pltpu.VMEM((RW,), jnp.float32),
        pltpu.VMEM((L,), jnp.float32),
        pltpu.SemaphoreType.DMA,
        pltpu.SemaphoreType.DMA,
    ],
)


def kernel(feat_idx, feat_val, feature_embed, linear_w, linear_b):
    idx3 = feat_idx.astype(jnp.int32).reshape(NCHUNKS, KB, 128)
    val2 = feat_val.astype(jnp.float32).reshape(NCHUNKS, IPC)
    bvec = jnp.broadcast_to(linear_b.astype(jnp.float32), (L,))
    return _FM(idx3, val2, feature_embed, linear_w, bvec)


# speedup vs baseline: 1.1007x; 1.0781x over previous
import jax
import jax.numpy as jnp
from jax import lax
from jax.experimental import pallas as pl
from jax.experimental.pallas import tpu as pltpu
from jax.experimental.pallas import tpu_sc as plsc

B = 16384
F = 26
D = 16
NC, NS, L = 2, 16, 16
NW = NC * NS
RW = B // NW
CH = 64
NCH = RW // CH
NT = NCH // 2
G = CH // L
IPC = F * CH
KB = IPC // 128
NCHUNKS = B // CH


def _fm_body(idx_hbm, val_hbm, emb_hbm, w_hbm, b_hbm, out_hbm,
             idx_l0, idx_l1, val_s0, val_s1, rows_0, rows_1,
             wv_0, wv_1, out_v, bv_v, sem0, sem1):
    wid = lax.axis_index("s") * NC + lax.axis_index("c")
    cid0 = wid * NCH
    pltpu.sync_copy(b_hbm, bv_v)
    iota = lax.iota(jnp.int32, L)
    bufs = ((idx_l0, val_s0, rows_0, wv_0, sem0),
            (idx_l1, val_s1, rows_1, wv_1, sem1))

    def stage(cid, buf):
        idx_l, val_s, rows_v, wv_v, sem = buf
        pltpu.sync_copy(idx_hbm.at[cid], idx_l)
        pltpu.sync_copy(val_hbm.at[cid], val_s)
        for k in range(KB):
            pltpu.async_copy(
                emb_hbm.at[idx_l.at[k]], rows_v.at[pl.ds(k * 128, 128)], sem)
            pltpu.async_copy(
                w_hbm.at[idx_l.at[k]], wv_v.at[pl.ds(k * 128, 128)], sem)

    def wait_streams(buf):
        idx_l, val_s, rows_v, wv_v, sem = buf
        for k in range(KB):
            pltpu.make_async_copy(
                emb_hbm.at[idx_l.at[k]], rows_v.at[pl.ds(k * 128, 128)],
                sem).wait()
            pltpu.make_async_copy(
                w_hbm.at[idx_l.at[k]], wv_v.at[pl.ds(k * 128, 128)],
                sem).wait()

    def compute(c, buf):
        _, val_s, rows_v, wv_v, _ = buf
        bvec = bv_v[...]
        for g in range(G):
            base = g * F
            tv = [val_s[pl.ds((base + j) * L, L)] for j in range(F)]
            l0 = bvec
            l1 = bvec * 0.0
            for j in range(F):
                wj = wv_v[pl.ds((base + j) * L, L)]
                if j % 2 == 0:
                    l0 = l0 + tv[j] * wj
                else:
                    l1 = l1 + tv[j] * wj
            lin = l0 + l1

            def d_body(d, acc, base=base, tv=tv):
                dvec = jnp.full((L,), d, dtype=jnp.int32)
                z = jnp.zeros((L,), jnp.float32)
                s0, s1, q0, q1 = z, z, z, z
                for j in range(F):
                    e = plsc.load_gather(
                        rows_v, [iota + (base + j) * L, dvec])
                    t = tv[j] * e
                    if j % 2 == 0:
                        s0 = s0 + t
                        q0 = q0 + t * t
                    else:
                        s1 = s1 + t
                        q1 = q1 + t * t
                s = s0 + s1
                return acc + (s * s - (q0 + q1))

            acc = lax.fori_loop(0, D, d_body, jnp.zeros((L,), jnp.float32))
            out_v[pl.ds(c * CH + g * L, L)] = lin + 0.5 * acc
        return

    stage(cid0, bufs[0])

    def body(t, carry):
        c0 = 2 * t
        stage(cid0 + c0 + 1, bufs[1])
        wait_streams(bufs[0])
        compute(c0, bufs[0])

        @pl.when(t < NT - 1)
        def _():
            stage(cid0 + c0 + 2, bufs[0])

        wait_streams(bufs[1])
        compute(c0 + 1, bufs[1])
        return carry

    lax.fori_loop(0, NT, body, 0)
    pltpu.sync_copy(out_v, out_hbm.at[pl.ds(wid * RW, RW)])


_MESH = plsc.VectorSubcoreMesh(
    core_axis_name="c", subcore_axis_name="s",
    num_cores=NC, num_subcores=NS)

_FM = pl.kernel(
    _fm_body,
    out_type=jax.ShapeDtypeStruct((B,), jnp.float32),
    mesh=_MESH,
    compiler_params=pltpu.CompilerParams(
        needs_layout_passes=False, use_tc_tiling_on_sc=False),
    scratch_types=[
        pltpu.VMEM((KB, 128), jnp.int32),
        pltpu.VMEM((KB, 128), jnp.int32),
        pltpu.VMEM((IPC,), jnp.float32),
        pltpu.VMEM((IPC,), jnp.float32),
        pltpu.VMEM((IPC, D), jnp.float32),
        pltpu.VMEM((IPC, D), jnp.float32),
        pltpu.VMEM((IPC,), jnp.float32),
        pltpu.VMEM((IPC,), jnp.float32),
        pltpu.VMEM((RW,), jnp.float32),
        pltpu.VMEM((L,), jnp.float32),
        pltpu.SemaphoreType.DMA,
        pltpu.SemaphoreType.DMA,
    ],
)


def kernel(feat_idx, feat_val, feature_embed, linear_w, linear_b):
    idx3 = (feat_idx.astype(jnp.int32).reshape(NCHUNKS, G, L, F)
            .transpose(0, 1, 3, 2).reshape(NCHUNKS, KB, 128))
    val2 = (feat_val.astype(jnp.float32).reshape(NCHUNKS, G, L, F)
            .transpose(0, 1, 3, 2).reshape(NCHUNKS, IPC))
    bvec = jnp.broadcast_to(linear_b.astype(jnp.float32), (L,))
    return _FM(idx3, val2, feature_embed, linear_w, bvec)
